# baseline (speedup 1.0000x reference)
import functools

import jax
import jax.numpy as jnp
from jax import lax
from jax.experimental import pallas as pl
from jax.experimental.pallas import tpu as pltpu
from jax.experimental.pallas import tpu_sc as plsc

NN = 10000
DIM = 128
EDGES = 320000
NTILE = 16
EPT = 20480
CH = 128
NCHUNK = EPT // CH
NBUF = 2
HALF = 5120
ACCR = HALF + 128
RPTZ = ACCR // 16
RPTO = HALF // 16
PADDST = 2 * HALF


def _tbl_body(x_ref, w_ref, b_ref, sgn_ref, out_ref):
    lin = lax.dot_general(
        x_ref[...], w_ref[...],
        dimension_numbers=(((1,), (1,)), ((), ())),
        preferred_element_type=jnp.float32,
    )
    out_ref[...] = sgn_ref[0] * (lin + b_ref[...])


def _build_table(x, W_w, b_row, sgn_b):
    bn = 2000
    nb = NN // bn
    return pl.pallas_call(
        _tbl_body,
        grid=(2, nb),
        in_specs=[
            pl.BlockSpec((bn, DIM), lambda a, j: (j, 0)),
            pl.BlockSpec((DIM, DIM), lambda a, j: (0, 0)),
            pl.BlockSpec((1, DIM), lambda a, j: (0, 0)),
            pl.BlockSpec((1, 1, DIM), lambda a, j: (a, 0, 0)),
        ],
        out_specs=pl.BlockSpec((bn, DIM), lambda a, j: (a * nb + j, 0)),
        out_shape=jax.ShapeDtypeStruct((2 * NN, DIM), jnp.float32),
    )(x, W_w, b_row, sgn_b)


def _idx_body(src_ref, attr_ref, out_ref):
    out_ref[...] = src_ref[...] + attr_ref[...] * NN


def _build_idx(src2, attr2):
    return pl.pallas_call(
        _idx_body,
        out_shape=jax.ShapeDtypeStruct(src2.shape, jnp.int32),
    )(src2, attr2)


_mesh = plsc.VectorSubcoreMesh(core_axis_name="c", subcore_axis_name="s")


@functools.partial(
    pl.kernel,
    out_type=jax.ShapeDtypeStruct((2, HALF, DIM), jnp.float32),
    mesh=_mesh,
    scratch_types=[
        pltpu.VMEM((NCHUNK, CH), jnp.int32),
        pltpu.VMEM((NCHUNK, CH), jnp.int32),
        [pltpu.VMEM((CH, DIM), jnp.float32)] * NBUF,
        [pltpu.SemaphoreType.DMA] * NBUF,
        pltpu.VMEM_SHARED((ACCR, DIM), jnp.float32),
    ],
)
def _sc_scatter(tbl_hbm, gidx_hbm, dst_hbm, zeros_hbm, out_hbm,
                gidx_v, dst_v, rows, gsems, acc_sh):
    c = lax.axis_index("c")
    s = lax.axis_index("s")

    pltpu.sync_copy(zeros_hbm.at[pl.ds(s * RPTZ, RPTZ)],
                    acc_sh.at[pl.ds(s * RPTZ, RPTZ)])
    pltpu.sync_copy(gidx_hbm.at[s], gidx_v)
    pltpu.sync_copy(dst_hbm.at[s], dst_v)

    base = c * HALF

    def remap(j, carry):
        for o in range(CH // 16):
            d = dst_v[j, pl.ds(o * 16, 16)] - base
            in_range = (d >= 0) & (d < HALF)
            dst_v[j, pl.ds(o * 16, 16)] = jnp.where(
                in_range, d, HALF + (d & 127))
        return carry

    lax.fori_loop(0, NCHUNK, remap, 0)
    plsc.subcore_barrier()

    def fire(j, k):
        pltpu.make_async_copy(tbl_hbm.at[gidx_v.at[j]], rows[k],
                              gsems[k]).start()

    def drain(j, k):
        pltpu.make_async_copy(tbl_hbm.at[gidx_v.at[j]], rows[k],
                              gsems[k]).wait()
        pltpu.sync_copy(rows[k], acc_sh.at[dst_v.at[j]], add=True)

    for k in range(NBUF - 1):
        fire(k, k)

    def ring_body(i, carry):
        for k in range(NBUF):
            j = NBUF * i + k

            @pl.when(j + NBUF - 1 < NCHUNK)
            def _():
                fire(j + NBUF - 1, (k + NBUF - 1) % NBUF)

            drain(j, k)
        return carry

    lax.fori_loop(0, NCHUNK // NBUF, ring_body, 0)
    plsc.subcore_barrier()
    pltpu.sync_copy(acc_sh.at[pl.ds(s * RPTO, RPTO)],
                    out_hbm.at[c, pl.ds(s * RPTO, RPTO)])


def _fin_body(sp_ref, x_ref, out_ref):
    out_ref[...] = x_ref[...] + jnp.maximum(sp_ref[...], 0.0)


def _finalize(s_cat, x):
    bn = 2000
    return pl.pallas_call(
        _fin_body,
        grid=(NN // bn,),
        in_specs=[
            pl.BlockSpec((bn, DIM), lambda j: (j, 0)),
            pl.BlockSpec((bn, DIM), lambda j: (j, 0)),
        ],
        out_specs=pl.BlockSpec((bn, DIM), lambda j: (j, 0)),
        out_shape=jax.ShapeDtypeStruct((NN, DIM), jnp.float32),
    )(s_cat, x)


def kernel(x, edge_index, edge_attr, W_w, W_b, sign_emb):
    npad = EPT - EDGES // NTILE
    src2 = jnp.concatenate(
        [edge_index[0].reshape(NTILE, EDGES // NTILE),
         jnp.zeros((NTILE, npad), jnp.int32)], axis=1).reshape(2560, 128)
    attr2 = jnp.concatenate(
        [edge_attr.astype(jnp.int32).reshape(NTILE, EDGES // NTILE),
         jnp.zeros((NTILE, npad), jnp.int32)], axis=1).reshape(2560, 128)
    pad_d = PADDST + (jnp.arange(npad, dtype=jnp.int32) & 127)
    dst_p = jnp.concatenate(
        [edge_index[1].reshape(NTILE, EDGES // NTILE),
         jnp.broadcast_to(pad_d, (NTILE, npad))], axis=1)
    sgn_b = jnp.broadcast_to(sign_emb[:, None, :], (2, 1, DIM))
    b_row = W_b.reshape(1, DIM)
    zeros = jnp.zeros((ACCR, DIM), jnp.float32)

    tbl = _build_table(x, W_w, b_row, sgn_b)
    gidx = _build_idx(src2, attr2).reshape(NTILE, NCHUNK, CH)
    dst_r = dst_p.reshape(NTILE, NCHUNK, CH)

    s_part = _sc_scatter(tbl, gidx, dst_r, zeros)
    s_cat = s_part.reshape(2 * HALF, DIM)
    return _finalize(s_cat, x)


# speedup vs baseline: 2.9810x; 2.9810x over previous
import functools

import jax
import jax.numpy as jnp
from jax import lax
from jax.experimental import pallas as pl
from jax.experimental.pallas import tpu as pltpu
from jax.experimental.pallas import tpu_sc as plsc

NN = 10000
DIM = 128
EDGES = 320000
NTILE = 16
EPT = 20000
CH = 80
NCHUNK = EPT // CH
NBUF = 2
HALF = 5120
ACCR = HALF + 128
RPTZ = ACCR // 16
RPTO = HALF // 16
PADDST = 2 * HALF


def _tbl_body(x_ref, w_ref, b_ref, sgn_ref, out_ref):
    lin = lax.dot_general(
        x_ref[...], w_ref[...],
        dimension_numbers=(((1,), (1,)), ((), ())),
        preferred_element_type=jnp.float32,
    )
    out_ref[...] = sgn_ref[0] * (lin + b_ref[...])


def _build_table(x, W_w, b_row, sgn_b):
    bn = 2000
    nb = NN // bn
    return pl.pallas_call(
        _tbl_body,
        grid=(2, nb),
        in_specs=[
            pl.BlockSpec((bn, DIM), lambda a, j: (j, 0)),
            pl.BlockSpec((DIM, DIM), lambda a, j: (0, 0)),
            pl.BlockSpec((1, DIM), lambda a, j: (0, 0)),
            pl.BlockSpec((1, 1, DIM), lambda a, j: (a, 0, 0)),
        ],
        out_specs=pl.BlockSpec((bn, DIM), lambda a, j: (a * nb + j, 0)),
        out_shape=jax.ShapeDtypeStruct((2 * NN, DIM), jnp.float32),
    )(x, W_w, b_row, sgn_b)


def _idx_body(src_ref, attr_ref, out_ref):
    out_ref[...] = src_ref[...] + attr_ref[...] * NN


def _build_idx(src2, attr2):
    return pl.pallas_call(
        _idx_body,
        out_shape=jax.ShapeDtypeStruct(src2.shape, jnp.int32),
    )(src2, attr2)


_mesh = plsc.VectorSubcoreMesh(core_axis_name="c", subcore_axis_name="s")


@functools.partial(
    pl.kernel,
    out_type=jax.ShapeDtypeStruct((2, HALF, DIM), jnp.float32),
    mesh=_mesh,
    scratch_types=[
        pltpu.VMEM((NCHUNK, CH), jnp.int32),
        pltpu.VMEM((NCHUNK, CH), jnp.int32),
        [pltpu.VMEM((CH, DIM), jnp.float32)] * NBUF,
        [pltpu.SemaphoreType.DMA] * NBUF,
        pltpu.VMEM_SHARED((ACCR, DIM), jnp.float32),
    ],
)
def _sc_scatter(tbl_hbm, gidx_hbm, dst_hbm, zeros_hbm, out_hbm,
                gidx_v, dst_v, rows, gsems, acc_sh):
    c = lax.axis_index("c")
    s = lax.axis_index("s")

    pltpu.sync_copy(zeros_hbm.at[pl.ds(s * RPTZ, RPTZ)],
                    acc_sh.at[pl.ds(s * RPTZ, RPTZ)])
    pltpu.sync_copy(gidx_hbm.at[s], gidx_v)
    pltpu.sync_copy(dst_hbm.at[s], dst_v)

    base = c * HALF

    def remap(j, carry):
        for o in range(CH // 16):
            d = dst_v[j, pl.ds(o * 16, 16)] - base
            in_range = (d >= 0) & (d < HALF)
            dst_v[j, pl.ds(o * 16, 16)] = jnp.where(
                in_range, d, HALF + (d & 127))
        return carry

    lax.fori_loop(0, NCHUNK, remap, 0)
    plsc.subcore_barrier()

    def fire(j, k):
        pltpu.make_async_copy(tbl_hbm.at[gidx_v.at[j]], rows[k],
                              gsems[k]).start()

    def drain(j, k):
        pltpu.make_async_copy(tbl_hbm.at[gidx_v.at[j]], rows[k],
                              gsems[k]).wait()
        pltpu.sync_copy(rows[k], acc_sh.at[dst_v.at[j]], add=True)

    for k in range(NBUF - 1):
        fire(k, k)

    def ring_body(i, carry):
        for k in range(NBUF):
            j = NBUF * i + k

            @pl.when(j + NBUF - 1 < NCHUNK)
            def _():
                fire(j + NBUF - 1, (k + NBUF - 1) % NBUF)

            drain(j, k)
        return carry

    lax.fori_loop(0, NCHUNK // NBUF, ring_body, 0)
    plsc.subcore_barrier()
    pltpu.sync_copy(acc_sh.at[pl.ds(s * RPTO, RPTO)],
                    out_hbm.at[c, pl.ds(s * RPTO, RPTO)])


def _fin_body(sp_ref, x_ref, out_ref):
    out_ref[...] = x_ref[...] + jnp.maximum(sp_ref[...], 0.0)


def _finalize(s_cat, x):
    bn = 2000
    return pl.pallas_call(
        _fin_body,
        grid=(NN // bn,),
        in_specs=[
            pl.BlockSpec((bn, DIM), lambda j: (j, 0)),
            pl.BlockSpec((bn, DIM), lambda j: (j, 0)),
        ],
        out_specs=pl.BlockSpec((bn, DIM), lambda j: (j, 0)),
        out_shape=jax.ShapeDtypeStruct((NN, DIM), jnp.float32),
    )(s_cat, x)


def kernel(x, edge_index, edge_attr, W_w, W_b, sign_emb):
    src2 = edge_index[0].reshape(2500, 128)
    attr2 = edge_attr.reshape(2500, 128).astype(jnp.int32)
    dst_p = edge_index[1]
    sgn_b = jnp.broadcast_to(sign_emb[:, None, :], (2, 1, DIM))
    b_row = W_b.reshape(1, DIM)
    zeros = jnp.zeros((ACCR, DIM), jnp.float32)

    tbl = _build_table(x, W_w, b_row, sgn_b)
    gidx = _build_idx(src2, attr2).reshape(NTILE, NCHUNK, CH)
    dst_r = dst_p.reshape(NTILE, NCHUNK, CH)

    s_part = _sc_scatter(tbl, gidx, dst_r, zeros)
    s_cat = s_part.reshape(2 * HALF, DIM)
    return _finalize(s_cat, x)


# speedup vs baseline: 2.9868x; 1.0019x over previous
import functools

import jax
import jax.numpy as jnp
from jax import lax
from jax.experimental import pallas as pl
from jax.experimental.pallas import tpu as pltpu
from jax.experimental.pallas import tpu_sc as plsc

NN = 10000
DIM = 128
EDGES = 320000
NTILE = 16
EPT = 20000
CH = 80
NCHUNK = EPT // CH
NBUF = 2
HALF = 5120
ACCR = HALF + 128
RPTZ = ACCR // 16
RPTO = HALF // 16
PADDST = 2 * HALF


def _pre_body(x_ref, w_ref, b_ref, sgn_ref, src_ref, attr_ref,
              tbl_ref, idx_ref):
    lin = lax.dot_general(
        x_ref[...], w_ref[...],
        dimension_numbers=(((1,), (1,)), ((), ())),
        preferred_element_type=jnp.float32,
    )
    tbl_ref[...] = sgn_ref[0] * (lin + b_ref[...])
    idx_ref[...] = src_ref[...] + attr_ref[...] * NN


def _build_pre(x, W_w, b_row, sgn_b, src2, attr2):
    bn = 2000
    nb = NN // bn
    erows = src2.shape[0] // (2 * nb)
    return pl.pallas_call(
        _pre_body,
        grid=(2, nb),
        in_specs=[
            pl.BlockSpec((bn, DIM), lambda a, j: (j, 0)),
            pl.BlockSpec((DIM, DIM), lambda a, j: (0, 0)),
            pl.BlockSpec((1, DIM), lambda a, j: (0, 0)),
            pl.BlockSpec((1, 1, DIM), lambda a, j: (a, 0, 0)),
            pl.BlockSpec((erows, 800), lambda a, j: (a * nb + j, 0)),
            pl.BlockSpec((erows, 800), lambda a, j: (a * nb + j, 0)),
        ],
        out_specs=[
            pl.BlockSpec((bn, DIM), lambda a, j: (a * nb + j, 0)),
            pl.BlockSpec((erows, 800), lambda a, j: (a * nb + j, 0)),
        ],
        out_shape=[
            jax.ShapeDtypeStruct((2 * NN, DIM), jnp.float32),
            jax.ShapeDtypeStruct(src2.shape, jnp.int32),
        ],
    )(x, W_w, b_row, sgn_b, src2, attr2)


_mesh = plsc.VectorSubcoreMesh(core_axis_name="c", subcore_axis_name="s")


@functools.partial(
    pl.kernel,
    out_type=jax.ShapeDtypeStruct((2, HALF, DIM), jnp.float32),
    mesh=_mesh,
    scratch_types=[
        pltpu.VMEM((NCHUNK, CH), jnp.int32),
        pltpu.VMEM((NCHUNK, CH), jnp.int32),
        [pltpu.VMEM((CH, DIM), jnp.float32)] * NBUF,
        [pltpu.SemaphoreType.DMA] * NBUF,
        pltpu.VMEM_SHARED((ACCR, DIM), jnp.float32),
    ],
)
def _sc_scatter(tbl_hbm, gidx_hbm, dst_hbm, zeros_hbm, out_hbm,
                gidx_v, dst_v, rows, gsems, acc_sh):
    c = lax.axis_index("c")
    s = lax.axis_index("s")

    pltpu.sync_copy(zeros_hbm.at[pl.ds(s * RPTZ, RPTZ)],
                    acc_sh.at[pl.ds(s * RPTZ, RPTZ)])
    pltpu.sync_copy(gidx_hbm.at[s], gidx_v)
    pltpu.sync_copy(dst_hbm.at[s], dst_v)

    base = c * HALF

    def remap(j, carry):
        for o in range(CH // 16):
            d = dst_v[j, pl.ds(o * 16, 16)] - base
            in_range = (d >= 0) & (d < HALF)
            dst_v[j, pl.ds(o * 16, 16)] = jnp.where(
                in_range, d, HALF + (d & 127))
        return carry

    lax.fori_loop(0, NCHUNK, remap, 0)
    plsc.subcore_barrier()

    def fire(j, k):
        pltpu.make_async_copy(tbl_hbm.at[gidx_v.at[j]], rows[k],
                              gsems[k]).start()

    def drain(j, k):
        pltpu.make_async_copy(tbl_hbm.at[gidx_v.at[j]], rows[k],
                              gsems[k]).wait()
        pltpu.sync_copy(rows[k], acc_sh.at[dst_v.at[j]], add=True)

    for k in range(NBUF - 1):
        fire(k, k)

    def ring_body(i, carry):
        for k in range(NBUF):
            j = NBUF * i + k

            @pl.when(j + NBUF - 1 < NCHUNK)
            def _():
                fire(j + NBUF - 1, (k + NBUF - 1) % NBUF)

            drain(j, k)
        return carry

    lax.fori_loop(0, NCHUNK // NBUF, ring_body, 0)
    plsc.subcore_barrier()
    pltpu.sync_copy(acc_sh.at[pl.ds(s * RPTO, RPTO)],
                    out_hbm.at[c, pl.ds(s * RPTO, RPTO)])


def _fin_body(sp_ref, x_ref, out_ref):
    out_ref[...] = x_ref[...] + jnp.maximum(sp_ref[...], 0.0)


def _finalize(s_cat, x):
    bn = 2000
    return pl.pallas_call(
        _fin_body,
        grid=(NN // bn,),
        in_specs=[
            pl.BlockSpec((bn, DIM), lambda j: (j, 0)),
            pl.BlockSpec((bn, DIM), lambda j: (j, 0)),
        ],
        out_specs=pl.BlockSpec((bn, DIM), lambda j: (j, 0)),
        out_shape=jax.ShapeDtypeStruct((NN, DIM), jnp.float32),
    )(s_cat, x)


def kernel(x, edge_index, edge_attr, W_w, W_b, sign_emb):
    src2 = edge_index[0].reshape(400, 800)
    attr2 = edge_attr.reshape(400, 800).astype(jnp.int32)
    dst_p = edge_index[1]
    sgn_b = jnp.broadcast_to(sign_emb[:, None, :], (2, 1, DIM))
    b_row = W_b.reshape(1, DIM)
    zeros = jnp.zeros((ACCR, DIM), jnp.float32)

    tbl, gidx = _build_pre(x, W_w, b_row, sgn_b, src2, attr2)
    gidx = gidx.reshape(NTILE, NCHUNK, CH)
    dst_r = dst_p.reshape(NTILE, NCHUNK, CH)

    s_part = _sc_scatter(tbl, gidx, dst_r, zeros)
    s_cat = s_part.reshape(2 * HALF, DIM)
    return _finalize(s_cat, x)


# speedup vs baseline: 3.0095x; 1.0076x over previous
import functools

import jax
import jax.numpy as jnp
from jax import lax
from jax.experimental import pallas as pl
from jax.experimental.pallas import tpu as pltpu
from jax.experimental.pallas import tpu_sc as plsc

NN = 10000
DIM = 128
EDGES = 320000
NTILE = 16
EPT = 20000
CH = 80
NCHUNK = EPT // CH
NBUF = 2
HALF = 5120
ACCR = HALF + 128
RPTZ = ACCR // 16
RPTO = HALF // 16
PADDST = 2 * HALF


def _pre_body(x_ref, w_ref, b_ref, sgn_ref, src_ref, attr_ref,
              tbl_ref, idx_ref):
    lin = lax.dot_general(
        x_ref[...], w_ref[...],
        dimension_numbers=(((1,), (1,)), ((), ())),
        preferred_element_type=jnp.float32,
    )
    tbl_ref[...] = sgn_ref[0] * (lin + b_ref[...])
    idx_ref[...] = src_ref[...] + attr_ref[...] * NN


def _build_pre(x, W_w, b_row, sgn_b, src2, attr2):
    bn = 2000
    nb = NN // bn
    erows = src2.shape[0] // (2 * nb)
    return pl.pallas_call(
        _pre_body,
        grid=(2, nb),
        in_specs=[
            pl.BlockSpec((bn, DIM), lambda a, j: (j, 0)),
            pl.BlockSpec((DIM, DIM), lambda a, j: (0, 0)),
            pl.BlockSpec((1, DIM), lambda a, j: (0, 0)),
            pl.BlockSpec((1, 1, DIM), lambda a, j: (a, 0, 0)),
            pl.BlockSpec((erows, 800), lambda a, j: (a * nb + j, 0)),
            pl.BlockSpec((erows, 800), lambda a, j: (a * nb + j, 0)),
        ],
        out_specs=[
            pl.BlockSpec((bn, DIM), lambda a, j: (a * nb + j, 0)),
            pl.BlockSpec((erows, 800), lambda a, j: (a * nb + j, 0)),
        ],
        out_shape=[
            jax.ShapeDtypeStruct((2 * NN, DIM), jnp.float32),
            jax.ShapeDtypeStruct(src2.shape, jnp.int32),
        ],
    )(x, W_w, b_row, sgn_b, src2, attr2)


_mesh = plsc.VectorSubcoreMesh(core_axis_name="c", subcore_axis_name="s")


@functools.partial(
    pl.kernel,
    out_type=jax.ShapeDtypeStruct((2, HALF, DIM), jnp.float32),
    mesh=_mesh,
    scratch_types=[
        pltpu.VMEM((NCHUNK, CH), jnp.int32),
        pltpu.VMEM((NCHUNK, CH), jnp.int32),
        [pltpu.VMEM((CH, DIM), jnp.float32)] * NBUF,
        [pltpu.SemaphoreType.DMA] * NBUF,
        pltpu.VMEM_SHARED((ACCR, DIM), jnp.float32),
    ],
)
def _sc_scatter(tbl_hbm, gidx_hbm, dst_hbm, zeros_hbm, out_hbm,
                gidx_v, dst_v, rows, gsems, acc_sh):
    c = lax.axis_index("c")
    s = lax.axis_index("s")

    pltpu.sync_copy(zeros_hbm.at[pl.ds(s * RPTZ, RPTZ)],
                    acc_sh.at[pl.ds(s * RPTZ, RPTZ)])
    pltpu.sync_copy(gidx_hbm.at[s], gidx_v)
    pltpu.sync_copy(dst_hbm.at[s], dst_v)

    base = c * HALF

    def remap(j):
        for o in range(CH // 16):
            d = dst_v[j, pl.ds(o * 16, 16)] - base
            in_range = (d >= 0) & (d < HALF)
            dst_v[j, pl.ds(o * 16, 16)] = jnp.where(
                in_range, d, HALF + (d & 127))

    plsc.subcore_barrier()

    def fire(j, k):
        pltpu.make_async_copy(tbl_hbm.at[gidx_v.at[j]], rows[k],
                              gsems[k]).start()

    def drain(j, k):
        pltpu.make_async_copy(tbl_hbm.at[gidx_v.at[j]], rows[k],
                              gsems[k]).wait()
        pltpu.sync_copy(rows[k], acc_sh.at[dst_v.at[j]], add=True)

    for k in range(NBUF - 1):
        fire(k, k)
        remap(k)

    def ring_body(i, carry):
        for k in range(NBUF):
            j = NBUF * i + k

            @pl.when(j + NBUF - 1 < NCHUNK)
            def _():
                fire(j + NBUF - 1, (k + NBUF - 1) % NBUF)
                remap(j + NBUF - 1)

            drain(j, k)
        return carry

    lax.fori_loop(0, NCHUNK // NBUF, ring_body, 0)
    plsc.subcore_barrier()
    pltpu.sync_copy(acc_sh.at[pl.ds(s * RPTO, RPTO)],
                    out_hbm.at[c, pl.ds(s * RPTO, RPTO)])


def _fin_body(sp_ref, x_ref, out_ref):
    out_ref[...] = x_ref[...] + jnp.maximum(sp_ref[...], 0.0)


def _finalize(s_cat, x):
    bn = 2000
    return pl.pallas_call(
        _fin_body,
        grid=(NN // bn,),
        in_specs=[
            pl.BlockSpec((bn, DIM), lambda j: (j, 0)),
            pl.BlockSpec((bn, DIM), lambda j: (j, 0)),
        ],
        out_specs=pl.BlockSpec((bn, DIM), lambda j: (j, 0)),
        out_shape=jax.ShapeDtypeStruct((NN, DIM), jnp.float32),
    )(s_cat, x)


def kernel(x, edge_index, edge_attr, W_w, W_b, sign_emb):
    src2 = edge_index[0].reshape(400, 800)
    attr2 = edge_attr.reshape(400, 800).astype(jnp.int32)
    dst_p = edge_index[1]
    sgn_b = jnp.broadcast_to(sign_emb[:, None, :], (2, 1, DIM))
    b_row = W_b.reshape(1, DIM)
    zeros = jnp.zeros((ACCR, DIM), jnp.float32)

    tbl, gidx = _build_pre(x, W_w, b_row, sgn_b, src2, attr2)
    gidx = gidx.reshape(NTILE, NCHUNK, CH)
    dst_r = dst_p.reshape(NTILE, NCHUNK, CH)

    s_part = _sc_scatter(tbl, gidx, dst_r, zeros)
    s_cat = s_part.reshape(2 * HALF, DIM)
    return _finalize(s_cat, x)
